# baseline (speedup 1.0000x reference)
import functools

import jax
import jax.numpy as jnp
from jax import lax
from jax.experimental import pallas as pl
from jax.experimental.pallas import tpu as pltpu
from jax.experimental.pallas import tpu_sc as plsc

N = 10000
E = 320000
D = 128
CH = 128
NCHUNK = E // CH
NC, NS = 2, 16
NW = NC * NS
NSLICE = 2
SLC = NCHUNK // NSLICE
ES = E // NSLICE
CPW = SLC // NW
REM_G = SLC - CPW * NW
CPS = SLC // NC
CPT = CPS // NS
REM_S = CPS - CPT * NS
RPT = N // NS

BE = 3200
BN = 2000


def _ln(u, g, beta):
    mu = jnp.mean(u, axis=-1, keepdims=True)
    d = u - mu
    var = jnp.mean(d * d, axis=-1, keepdims=True)
    return d / jnp.sqrt(var + 1e-5) * g + beta


def _silu(x):
    return x * jax.nn.sigmoid(x)


def _mlp1_body(x_ref, w1_ref, b1_ref, w2_ref, b2_ref, g_ref, bt_ref, o_ref):
    h = _silu(jnp.dot(x_ref[...], w1_ref[...],
                      preferred_element_type=jnp.float32) + b1_ref[...])
    u = jnp.dot(h, w2_ref[...], preferred_element_type=jnp.float32) + b2_ref[...]
    o_ref[...] = _ln(u, g_ref[...], bt_ref[...]).astype(o_ref.dtype)


def _mlp1(x, p, rb, out_dtype=jnp.float32):
    r, k = x.shape
    vec = lambda a: a.reshape(1, -1)
    return pl.pallas_call(
        _mlp1_body,
        grid=(r // rb,),
        in_specs=[
            pl.BlockSpec((rb, k), lambda i: (i, 0)),
            pl.BlockSpec((k, D), lambda i: (0, 0)),
            pl.BlockSpec((1, D), lambda i: (0, 0)),
            pl.BlockSpec((D, D), lambda i: (0, 0)),
            pl.BlockSpec((1, D), lambda i: (0, 0)),
            pl.BlockSpec((1, D), lambda i: (0, 0)),
            pl.BlockSpec((1, D), lambda i: (0, 0)),
        ],
        out_specs=pl.BlockSpec((rb, D), lambda i: (i, 0)),
        out_shape=jax.ShapeDtypeStruct((r, D), out_dtype),
    )(x, p['w1'], vec(p['b1']), p['w2'], vec(p['b2']), vec(p['g']), vec(p['beta']))


def _proj2_body(h_ref, wa_ref, wb_ref, a_ref, b_ref):
    h = h_ref[...]
    a_ref[...] = jnp.dot(h, wa_ref[...], preferred_element_type=jnp.float32)
    b_ref[...] = jnp.dot(h, wb_ref[...], preferred_element_type=jnp.float32)


def _proj2(h, wa, wb):
    return pl.pallas_call(
        _proj2_body,
        grid=(N // BN,),
        in_specs=[
            pl.BlockSpec((BN, D), lambda i: (i, 0)),
            pl.BlockSpec((D, D), lambda i: (0, 0)),
            pl.BlockSpec((D, D), lambda i: (0, 0)),
        ],
        out_specs=[
            pl.BlockSpec((BN, D), lambda i: (i, 0)),
            pl.BlockSpec((BN, D), lambda i: (i, 0)),
        ],
        out_shape=[
            jax.ShapeDtypeStruct((N, D), jnp.float32),
            jax.ShapeDtypeStruct((N, D), jnp.float32),
        ],
    )(h, wa, wb)


def _edge_body(pre_ref, he_ref, wc_ref, b1_ref, w2_ref, b2_ref,
               g_ref, bt_ref, o_ref):
    u = (pre_ref[...]
         + jnp.dot(he_ref[...], wc_ref[...], preferred_element_type=jnp.float32)
         + b1_ref[...])
    u = _silu(u)
    v = jnp.dot(u, w2_ref[...], preferred_element_type=jnp.float32) + b2_ref[...]
    o_ref[...] = _ln(v, g_ref[...], bt_ref[...])


def _edge_mlp(pre, he, wc, p, s):
    vec = lambda a: a.reshape(1, -1)
    mat = lambda: pl.BlockSpec((D, D), lambda i: (0, 0))
    one = lambda: pl.BlockSpec((1, D), lambda i: (0, 0))
    gs = ES // BE
    return pl.pallas_call(
        _edge_body,
        grid=(gs,),
        in_specs=[pl.BlockSpec((BE, D), lambda i: (i, 0)),
                  pl.BlockSpec((BE, D), lambda i, s=s: (s * gs + i, 0)),
                  mat(), one(), mat(), one(), one(), one()],
        out_specs=pl.BlockSpec((BE, D), lambda i: (i, 0)),
        out_shape=jax.ShapeDtypeStruct((ES, D), jnp.float32),
    )(pre, he, wc, vec(p['b1']), p['w2'], vec(p['b2']),
      vec(p['g']), vec(p['beta']))


def _node_body(h_ref, p0_ref, p1_ref, p2_ref, p3_ref, w1h_ref, w1a_ref,
               b1_ref, w2_ref, b2_ref, g_ref, bt_ref, o_ref):
    h = h_ref[...]
    agg = (p0_ref[...] + p1_ref[...]) + (p2_ref[...] + p3_ref[...])
    u = (jnp.dot(h, w1h_ref[...], preferred_element_type=jnp.float32)
         + jnp.dot(agg, w1a_ref[...], preferred_element_type=jnp.float32)
         + b1_ref[...])
    u = _silu(u)
    v = jnp.dot(u, w2_ref[...], preferred_element_type=jnp.float32) + b2_ref[...]
    o_ref[...] = h + _ln(v, g_ref[...], bt_ref[...])


def _node_mlp(h, parts, p):
    vec = lambda a: a.reshape(1, -1)
    row = lambda: pl.BlockSpec((BN, D), lambda i: (i, 0))
    mat = lambda: pl.BlockSpec((D, D), lambda i: (0, 0))
    one = lambda: pl.BlockSpec((1, D), lambda i: (0, 0))
    w1 = p['w1']
    return pl.pallas_call(
        _node_body,
        grid=(N // BN,),
        in_specs=[row(), row(), row(), row(), row(),
                  mat(), mat(), one(), mat(), one(), one(), one()],
        out_specs=pl.BlockSpec((BN, D), lambda i: (i, 0)),
        out_shape=jax.ShapeDtypeStruct((N, D), jnp.float32),
    )(h, parts[0], parts[1], parts[2], parts[3], w1[:D], w1[D:],
      vec(p['b1']), p['w2'], vec(p['b2']), vec(p['g']), vec(p['beta']))


def _dec_body(h_ref, w1_ref, b1_ref, w2_ref, b2_ref, o_ref):
    u = _silu(jnp.dot(h_ref[...], w1_ref[...],
                      preferred_element_type=jnp.float32) + b1_ref[...])
    o_ref[...] = jnp.dot(u, w2_ref[...], preferred_element_type=jnp.float32) + b2_ref[...]


def _decoder(h, w1, b1, w2p, b2p):
    hd = w1.shape[1]
    return pl.pallas_call(
        _dec_body,
        grid=(N // BN,),
        in_specs=[
            pl.BlockSpec((BN, D), lambda i: (i, 0)),
            pl.BlockSpec((D, hd), lambda i: (0, 0)),
            pl.BlockSpec((1, hd), lambda i: (0, 0)),
            pl.BlockSpec((hd, D), lambda i: (0, 0)),
            pl.BlockSpec((1, D), lambda i: (0, 0)),
        ],
        out_specs=pl.BlockSpec((BN, D), lambda i: (i, 0)),
        out_shape=jax.ShapeDtypeStruct((N, D), jnp.float32),
    )(h, w1, b1.reshape(1, -1), w2p, b2p.reshape(1, -1))


@functools.cache
def _build_sc_gather2(c0):
    NB = 3

    @functools.partial(
        pl.kernel,
        out_type=jax.ShapeDtypeStruct((ES, D), jnp.float32),
        scratch_types=[
            pltpu.VMEM((NB, CH), jnp.int32),
            pltpu.VMEM((NB, CH), jnp.int32),
            pltpu.VMEM((NB, CH, D), jnp.float32),
            pltpu.VMEM((NB, CH, D), jnp.float32),
        ] + [pltpu.SemaphoreType.DMA] * (3 * NB),
        mesh=plsc.VectorSubcoreMesh(core_axis_name="c", subcore_axis_name="s"),
    )
    def _sc_gather2(ha_hbm, hb_hbm, dst_hbm, src_hbm, pre_hbm,
                    idxd, idxs, rowsa, rowsb, *sems):
        wid = lax.axis_index("s") * NC + lax.axis_index("c")
        base = wid * CPW
        semi = sems[0:NB]
        semg = sems[NB:2 * NB]
        semo = sems[2 * NB:3 * NB]

        def issue_idx(b, lc):
            pltpu.async_copy(dst_hbm.at[c0 + lc], idxd.at[b], semi[b])
            pltpu.async_copy(src_hbm.at[c0 + lc], idxs.at[b], semi[b])

        def wait_idx(b):
            pltpu.make_async_copy(dst_hbm.at[0], idxd.at[b], semi[b]).wait()
            pltpu.make_async_copy(dst_hbm.at[0], idxs.at[b], semi[b]).wait()

        def issue_gather(b):
            pltpu.async_copy(ha_hbm.at[idxd.at[b]], rowsa.at[b], semg[b])
            pltpu.async_copy(hb_hbm.at[idxs.at[b]], rowsb.at[b], semg[b])

        def wait_gather(b):
            pltpu.make_async_copy(ha_hbm.at[idxd.at[b]], rowsa.at[b],
                                  semg[b]).wait()
            pltpu.make_async_copy(ha_hbm.at[idxd.at[b]], rowsb.at[b],
                                  semg[b]).wait()

        def add_rows(b):
            def row(r, carry):
                for j in range(D // 16):
                    sl = pl.ds(j * 16, 16)
                    rowsa[b, r, sl] += rowsb[b, r, sl]
                return carry
            lax.fori_loop(0, CH, row, 0)

        def issue_out(b, lc):
            off = pl.multiple_of(lc * CH, CH)
            pltpu.async_copy(rowsa.at[b], pre_hbm.at[pl.ds(off, CH)], semo[b])

        def wait_out(b):
            pltpu.make_async_copy(rowsa.at[b], pre_hbm.at[pl.ds(0, CH)],
                                  semo[b]).wait()

        def body(j, carry):
            cb = base + NB * j
            for b in range(NB):
                issue_idx(b, cb + b)
            for b in range(NB):
                wait_idx(b)

                @pl.when(j > 0)
                def _(b=b):
                    wait_out(b)

                issue_gather(b)
            for b in range(NB):
                wait_gather(b)
                add_rows(b)
                issue_out(b, cb + b)
            return carry

        lax.fori_loop(0, CPW // NB, body, 0)
        for b in range(NB):
            wait_out(b)

        @pl.when(wid < REM_G)
        def _():
            lc = NW * CPW + wid
            issue_idx(0, lc)
            wait_idx(0)
            issue_gather(0)
            wait_gather(0)
            add_rows(0)
            issue_out(0, lc)
            wait_out(0)

    return _sc_gather2


@functools.cache
def _build_sc_scatter(c0):
    @functools.partial(
        pl.kernel,
        out_type=jax.ShapeDtypeStruct((NC, N, D), jnp.float32),
        scratch_types=[
            pltpu.VMEM((3, CH), jnp.int32),
            pltpu.VMEM((3, CH, D), jnp.float32),
            pltpu.VMEM_SHARED((N, D), jnp.float32),
        ] + [pltpu.SemaphoreType.DMA] * 9,
        mesh=plsc.VectorSubcoreMesh(core_axis_name="c", subcore_axis_name="s"),
    )
    def _sc_scatter(m_hbm, dst_hbm, out_hbm, idx, rows, acc, *sems9):
        cid = lax.axis_index("c")
        sid = lax.axis_index("s")
        semi = sems9[0:3]
        semr = sems9[3:6]
        sems = sems9[6:9]

        def zrow(r, carry):
            for j in range(D // 16):
                rows[0, r, pl.ds(j * 16, 16)] = jnp.zeros((16,), jnp.float32)
            return carry

        lax.fori_loop(0, CH, zrow, 0)

        start = pl.multiple_of((RPT * sid) // 8 * 8, 8)
        end = jnp.where(sid == NS - 1, N, (RPT * (sid + 1)) // 8 * 8)

        def over_stripe(fn):
            for t in range(4):
                fn(pl.multiple_of(start + t * CH, 8), CH)
            tail = end - start - 4 * CH

            @pl.when(tail == 112)
            def _():
                fn(pl.multiple_of(start + 4 * CH, 8), 112)

            @pl.when(tail == 120)
            def _():
                fn(pl.multiple_of(start + 4 * CH, 8), 120)

        over_stripe(lambda off, n: pltpu.sync_copy(
            rows.at[0, pl.ds(0, n)], acc.at[pl.ds(off, n)]))
        plsc.subcore_barrier()

        base = cid * CPS + sid * CPT

        def issue_load(b, lc):
            off = pl.multiple_of(lc * CH, CH)
            pltpu.async_copy(dst_hbm.at[c0 + lc], idx.at[b], semi[b])
            pltpu.async_copy(m_hbm.at[pl.ds(off, CH)], rows.at[b], semr[b])

        def wait_load(b):
            pltpu.make_async_copy(dst_hbm.at[0], idx.at[b], semi[b]).wait()
            pltpu.make_async_copy(m_hbm.at[pl.ds(0, CH)], rows.at[b],
                                  semr[b]).wait()

        def issue_scatter(b):
            pltpu.async_copy(rows.at[b], acc.at[idx.at[b]], sems[b], add=True)

        def wait_scatter(b):
            pltpu.make_async_copy(rows.at[b], acc.at[idx.at[b]],
                                  sems[b]).wait()

        def body(k, carry):
            lc = base + 3 * k
            for b in range(3):
                @pl.when(k > 0)
                def _(b=b):
                    wait_scatter(b)

                issue_load(b, lc + b)
            for b in range(3):
                wait_load(b)
                issue_scatter(b)
            return carry

        lax.fori_loop(0, CPT // 3, body, 0)
        for b in range(3):
            wait_scatter(b)

        @pl.when(sid < REM_S)
        def _():
            lc = cid * CPS + NS * CPT + sid
            issue_load(0, lc)
            wait_load(0)
            issue_scatter(0)
            wait_scatter(0)

        plsc.subcore_barrier()
        over_stripe(lambda off, n: pltpu.sync_copy(
            acc.at[pl.ds(off, n)], out_hbm.at[cid, pl.ds(off, n)]))

    return _sc_scatter


def kernel(x, edge_index, edge_attr, params):
    dst = edge_index[1].astype(jnp.int32).reshape(NCHUNK, CH)
    src = edge_index[0].astype(jnp.int32).reshape(NCHUNK, CH)

    h = _mlp1(x, params['node_enc'], BN)
    he = _mlp1(edge_attr, params['edge_enc'], BE)

    for blk in params['blocks']:
        ew = blk['edge_mlp']
        w1 = ew['w1']
        ha, hb = _proj2(h, w1[:D], w1[D:2 * D])
        parts = []
        ms = []
        for s in range(NSLICE):
            pre = _build_sc_gather2(s * SLC)(ha, hb, dst, src)
            ms.append(_edge_mlp(pre, he, w1[2 * D:], ew, s))
        for s in range(NSLICE):
            p = _build_sc_scatter(s * SLC)(ms[s], dst)
            parts += [p[0], p[1]]
        h = _node_mlp(h, parts, blk['node_mlp'])

    w2p = jnp.pad(params['dec_w2'], ((0, 0), (0, D - params['dec_w2'].shape[1])))
    b2p = jnp.pad(params['dec_b2'], (0, D - params['dec_b2'].shape[0]))
    out = _decoder(h, params['dec_w1'], params['dec_b1'], w2p, b2p)
    return out[:, :params['dec_w2'].shape[1]]


# speedup vs baseline: 1.0428x; 1.0428x over previous
import functools

import jax
import jax.numpy as jnp
from jax import lax
from jax.experimental import pallas as pl
from jax.experimental.pallas import tpu as pltpu
from jax.experimental.pallas import tpu_sc as plsc

N = 10000
E = 320000
D = 128
CH = 128
NCHUNK = E // CH
NC, NS = 2, 16
NW = NC * NS
NSLICE = 2
SLC = NCHUNK // NSLICE
ES = E // NSLICE
CPW = SLC // NW
REM_G = SLC - CPW * NW
CPS = SLC // NC
CPT = CPS // NS
REM_S = CPS - CPT * NS
RPT = N // NS

BE = 3200
BN = 2000


def _ln(u, g, beta):
    mu = jnp.mean(u, axis=-1, keepdims=True)
    d = u - mu
    var = jnp.mean(d * d, axis=-1, keepdims=True)
    return d / jnp.sqrt(var + 1e-5) * g + beta


def _silu(x):
    return x * jax.nn.sigmoid(x)


def _mlp1_body(x_ref, w1_ref, b1_ref, w2_ref, b2_ref, g_ref, bt_ref, o_ref):
    h = _silu(jnp.dot(x_ref[...], w1_ref[...],
                      preferred_element_type=jnp.float32) + b1_ref[...])
    u = jnp.dot(h, w2_ref[...], preferred_element_type=jnp.float32) + b2_ref[...]
    o_ref[...] = _ln(u, g_ref[...], bt_ref[...]).astype(o_ref.dtype)


def _mlp1(x, p, rb, out_dtype=jnp.float32):
    r, k = x.shape
    vec = lambda a: a.reshape(1, -1)
    return pl.pallas_call(
        _mlp1_body,
        grid=(r // rb,),
        in_specs=[
            pl.BlockSpec((rb, k), lambda i: (i, 0)),
            pl.BlockSpec((k, D), lambda i: (0, 0)),
            pl.BlockSpec((1, D), lambda i: (0, 0)),
            pl.BlockSpec((D, D), lambda i: (0, 0)),
            pl.BlockSpec((1, D), lambda i: (0, 0)),
            pl.BlockSpec((1, D), lambda i: (0, 0)),
            pl.BlockSpec((1, D), lambda i: (0, 0)),
        ],
        out_specs=pl.BlockSpec((rb, D), lambda i: (i, 0)),
        out_shape=jax.ShapeDtypeStruct((r, D), out_dtype),
    )(x, p['w1'], vec(p['b1']), p['w2'], vec(p['b2']), vec(p['g']), vec(p['beta']))


def _proj2_body(h_ref, wa_ref, wb_ref, a_ref, b_ref):
    h = h_ref[...]
    a_ref[...] = jnp.dot(h, wa_ref[...], preferred_element_type=jnp.float32)
    b_ref[...] = jnp.dot(h, wb_ref[...], preferred_element_type=jnp.float32)


def _proj2(h, wa, wb):
    return pl.pallas_call(
        _proj2_body,
        grid=(N // BN,),
        in_specs=[
            pl.BlockSpec((BN, D), lambda i: (i, 0)),
            pl.BlockSpec((D, D), lambda i: (0, 0)),
            pl.BlockSpec((D, D), lambda i: (0, 0)),
        ],
        out_specs=[
            pl.BlockSpec((BN, D), lambda i: (i, 0)),
            pl.BlockSpec((BN, D), lambda i: (i, 0)),
        ],
        out_shape=[
            jax.ShapeDtypeStruct((N, D), jnp.float32),
            jax.ShapeDtypeStruct((N, D), jnp.float32),
        ],
    )(h, wa, wb)


def _edge_body(pre_ref, he_ref, wc_ref, b1_ref, w2_ref, b2_ref,
               g_ref, bt_ref, o_ref):
    u = (pre_ref[...]
         + jnp.dot(he_ref[...], wc_ref[...], preferred_element_type=jnp.float32)
         + b1_ref[...])
    u = _silu(u).astype(jnp.bfloat16)
    v = jnp.dot(u, w2_ref[...], preferred_element_type=jnp.float32) + b2_ref[...]
    o_ref[...] = _ln(v, g_ref[...], bt_ref[...])


def _edge_mlp(pre, he, wc, p, s):
    vec = lambda a: a.reshape(1, -1)
    mat = lambda: pl.BlockSpec((D, D), lambda i: (0, 0))
    one = lambda: pl.BlockSpec((1, D), lambda i: (0, 0))
    gs = ES // BE
    return pl.pallas_call(
        _edge_body,
        grid=(gs,),
        in_specs=[pl.BlockSpec((BE, D), lambda i: (i, 0)),
                  pl.BlockSpec((BE, D), lambda i, s=s: (s * gs + i, 0)),
                  mat(), one(), mat(), one(), one(), one()],
        out_specs=pl.BlockSpec((BE, D), lambda i: (i, 0)),
        out_shape=jax.ShapeDtypeStruct((ES, D), jnp.float32),
    )(pre, he, wc.astype(jnp.bfloat16), vec(p['b1']),
      p['w2'].astype(jnp.bfloat16), vec(p['b2']), vec(p['g']), vec(p['beta']))


def _node_body(h_ref, p0_ref, p1_ref, p2_ref, p3_ref, w1h_ref, w1a_ref,
               b1_ref, w2_ref, b2_ref, g_ref, bt_ref, o_ref):
    h = h_ref[...]
    agg = (p0_ref[...] + p1_ref[...]) + (p2_ref[...] + p3_ref[...])
    u = (jnp.dot(h, w1h_ref[...], preferred_element_type=jnp.float32)
         + jnp.dot(agg, w1a_ref[...], preferred_element_type=jnp.float32)
         + b1_ref[...])
    u = _silu(u)
    v = jnp.dot(u, w2_ref[...], preferred_element_type=jnp.float32) + b2_ref[...]
    o_ref[...] = h + _ln(v, g_ref[...], bt_ref[...])


def _node_mlp(h, parts, p):
    vec = lambda a: a.reshape(1, -1)
    row = lambda: pl.BlockSpec((BN, D), lambda i: (i, 0))
    mat = lambda: pl.BlockSpec((D, D), lambda i: (0, 0))
    one = lambda: pl.BlockSpec((1, D), lambda i: (0, 0))
    w1 = p['w1']
    return pl.pallas_call(
        _node_body,
        grid=(N // BN,),
        in_specs=[row(), row(), row(), row(), row(),
                  mat(), mat(), one(), mat(), one(), one(), one()],
        out_specs=pl.BlockSpec((BN, D), lambda i: (i, 0)),
        out_shape=jax.ShapeDtypeStruct((N, D), jnp.float32),
    )(h, parts[0], parts[1], parts[2], parts[3], w1[:D], w1[D:],
      vec(p['b1']), p['w2'], vec(p['b2']), vec(p['g']), vec(p['beta']))


def _dec_body(h_ref, w1_ref, b1_ref, w2_ref, b2_ref, o_ref):
    u = _silu(jnp.dot(h_ref[...], w1_ref[...],
                      preferred_element_type=jnp.float32) + b1_ref[...])
    o_ref[...] = jnp.dot(u, w2_ref[...], preferred_element_type=jnp.float32) + b2_ref[...]


def _decoder(h, w1, b1, w2p, b2p):
    hd = w1.shape[1]
    return pl.pallas_call(
        _dec_body,
        grid=(N // BN,),
        in_specs=[
            pl.BlockSpec((BN, D), lambda i: (i, 0)),
            pl.BlockSpec((D, hd), lambda i: (0, 0)),
            pl.BlockSpec((1, hd), lambda i: (0, 0)),
            pl.BlockSpec((hd, D), lambda i: (0, 0)),
            pl.BlockSpec((1, D), lambda i: (0, 0)),
        ],
        out_specs=pl.BlockSpec((BN, D), lambda i: (i, 0)),
        out_shape=jax.ShapeDtypeStruct((N, D), jnp.float32),
    )(h, w1, b1.reshape(1, -1), w2p, b2p.reshape(1, -1))


@functools.cache
def _build_sc_gather2(c0):
    NB = 3

    @functools.partial(
        pl.kernel,
        out_type=jax.ShapeDtypeStruct((ES, D), jnp.float32),
        scratch_types=[
            pltpu.VMEM((NB, CH), jnp.int32),
            pltpu.VMEM((NB, CH), jnp.int32),
            pltpu.VMEM((NB, CH, D), jnp.float32),
            pltpu.VMEM((NB, CH, D), jnp.float32),
        ] + [pltpu.SemaphoreType.DMA] * (3 * NB),
        mesh=plsc.VectorSubcoreMesh(core_axis_name="c", subcore_axis_name="s"),
    )
    def _sc_gather2(ha_hbm, hb_hbm, dst_hbm, src_hbm, pre_hbm,
                    idxd, idxs, rowsa, rowsb, *sems):
        wid = lax.axis_index("s") * NC + lax.axis_index("c")
        base = wid * CPW
        semi = sems[0:NB]
        semg = sems[NB:2 * NB]
        semo = sems[2 * NB:3 * NB]

        def issue_idx(b, lc):
            pltpu.async_copy(dst_hbm.at[c0 + lc], idxd.at[b], semi[b])
            pltpu.async_copy(src_hbm.at[c0 + lc], idxs.at[b], semi[b])

        def wait_idx(b):
            pltpu.make_async_copy(dst_hbm.at[0], idxd.at[b], semi[b]).wait()
            pltpu.make_async_copy(dst_hbm.at[0], idxs.at[b], semi[b]).wait()

        def issue_gather(b):
            pltpu.async_copy(ha_hbm.at[idxd.at[b]], rowsa.at[b], semg[b])
            pltpu.async_copy(hb_hbm.at[idxs.at[b]], rowsb.at[b], semg[b])

        def wait_gather(b):
            pltpu.make_async_copy(ha_hbm.at[idxd.at[b]], rowsa.at[b],
                                  semg[b]).wait()
            pltpu.make_async_copy(ha_hbm.at[idxd.at[b]], rowsb.at[b],
                                  semg[b]).wait()

        def add_rows(b):
            def row(r, carry):
                for j in range(D // 16):
                    sl = pl.ds(j * 16, 16)
                    rowsa[b, r, sl] += rowsb[b, r, sl]
                return carry
            lax.fori_loop(0, CH, row, 0)

        def issue_out(b, lc):
            off = pl.multiple_of(lc * CH, CH)
            pltpu.async_copy(rowsa.at[b], pre_hbm.at[pl.ds(off, CH)], semo[b])

        def wait_out(b):
            pltpu.make_async_copy(rowsa.at[b], pre_hbm.at[pl.ds(0, CH)],
                                  semo[b]).wait()

        def body(j, carry):
            cb = base + NB * j
            for b in range(NB):
                issue_idx(b, cb + b)
            for b in range(NB):
                wait_idx(b)

                @pl.when(j > 0)
                def _(b=b):
                    wait_out(b)

                issue_gather(b)
            for b in range(NB):
                wait_gather(b)
                add_rows(b)
                issue_out(b, cb + b)
            return carry

        lax.fori_loop(0, CPW // NB, body, 0)
        for b in range(NB):
            wait_out(b)

        @pl.when(wid < REM_G)
        def _():
            lc = NW * CPW + wid
            issue_idx(0, lc)
            wait_idx(0)
            issue_gather(0)
            wait_gather(0)
            add_rows(0)
            issue_out(0, lc)
            wait_out(0)

    return _sc_gather2


@functools.cache
def _build_sc_scatter(c0):
    @functools.partial(
        pl.kernel,
        out_type=jax.ShapeDtypeStruct((NC, N, D), jnp.float32),
        scratch_types=[
            pltpu.VMEM((3, CH), jnp.int32),
            pltpu.VMEM((3, CH, D), jnp.float32),
            pltpu.VMEM_SHARED((N, D), jnp.float32),
        ] + [pltpu.SemaphoreType.DMA] * 9,
        mesh=plsc.VectorSubcoreMesh(core_axis_name="c", subcore_axis_name="s"),
    )
    def _sc_scatter(m_hbm, dst_hbm, out_hbm, idx, rows, acc, *sems9):
        cid = lax.axis_index("c")
        sid = lax.axis_index("s")
        semi = sems9[0:3]
        semr = sems9[3:6]
        sems = sems9[6:9]

        def zrow(r, carry):
            for j in range(D // 16):
                rows[0, r, pl.ds(j * 16, 16)] = jnp.zeros((16,), jnp.float32)
            return carry

        lax.fori_loop(0, CH, zrow, 0)

        start = pl.multiple_of((RPT * sid) // 8 * 8, 8)
        end = jnp.where(sid == NS - 1, N, (RPT * (sid + 1)) // 8 * 8)

        def over_stripe(fn):
            for t in range(4):
                fn(pl.multiple_of(start + t * CH, 8), CH)
            tail = end - start - 4 * CH

            @pl.when(tail == 112)
            def _():
                fn(pl.multiple_of(start + 4 * CH, 8), 112)

            @pl.when(tail == 120)
            def _():
                fn(pl.multiple_of(start + 4 * CH, 8), 120)

        over_stripe(lambda off, n: pltpu.sync_copy(
            rows.at[0, pl.ds(0, n)], acc.at[pl.ds(off, n)]))
        plsc.subcore_barrier()

        base = cid * CPS + sid * CPT

        def issue_load(b, lc):
            off = pl.multiple_of(lc * CH, CH)
            pltpu.async_copy(dst_hbm.at[c0 + lc], idx.at[b], semi[b])
            pltpu.async_copy(m_hbm.at[pl.ds(off, CH)], rows.at[b], semr[b])

        def wait_load(b):
            pltpu.make_async_copy(dst_hbm.at[0], idx.at[b], semi[b]).wait()
            pltpu.make_async_copy(m_hbm.at[pl.ds(0, CH)], rows.at[b],
                                  semr[b]).wait()

        def issue_scatter(b):
            pltpu.async_copy(rows.at[b], acc.at[idx.at[b]], sems[b], add=True)

        def wait_scatter(b):
            pltpu.make_async_copy(rows.at[b], acc.at[idx.at[b]],
                                  sems[b]).wait()

        def body(k, carry):
            lc = base + 3 * k
            for b in range(3):
                @pl.when(k > 0)
                def _(b=b):
                    wait_scatter(b)

                issue_load(b, lc + b)
            for b in range(3):
                wait_load(b)
                issue_scatter(b)
            return carry

        lax.fori_loop(0, CPT // 3, body, 0)
        for b in range(3):
            wait_scatter(b)

        @pl.when(sid < REM_S)
        def _():
            lc = cid * CPS + NS * CPT + sid
            issue_load(0, lc)
            wait_load(0)
            issue_scatter(0)
            wait_scatter(0)

        plsc.subcore_barrier()
        over_stripe(lambda off, n: pltpu.sync_copy(
            acc.at[pl.ds(off, n)], out_hbm.at[cid, pl.ds(off, n)]))

    return _sc_scatter


def kernel(x, edge_index, edge_attr, params):
    dst = edge_index[1].astype(jnp.int32).reshape(NCHUNK, CH)
    src = edge_index[0].astype(jnp.int32).reshape(NCHUNK, CH)

    h = _mlp1(x, params['node_enc'], BN)
    he = _mlp1(edge_attr, params['edge_enc'], BE, out_dtype=jnp.bfloat16)

    for blk in params['blocks']:
        ew = blk['edge_mlp']
        w1 = ew['w1']
        ha, hb = _proj2(h, w1[:D], w1[D:2 * D])
        parts = []
        ms = []
        for s in range(NSLICE):
            pre = _build_sc_gather2(s * SLC)(ha, hb, dst, src)
            ms.append(_edge_mlp(pre, he, w1[2 * D:], ew, s))
        for s in range(NSLICE):
            p = _build_sc_scatter(s * SLC)(ms[s], dst)
            parts += [p[0], p[1]]
        h = _node_mlp(h, parts, blk['node_mlp'])

    w2p = jnp.pad(params['dec_w2'], ((0, 0), (0, D - params['dec_w2'].shape[1])))
    b2p = jnp.pad(params['dec_b2'], (0, D - params['dec_b2'].shape[0]))
    out = _decoder(h, params['dec_w1'], params['dec_b1'], w2p, b2p)
    return out[:, :params['dec_w2'].shape[1]]
